# baseline (speedup 1.0000x reference)
import jax
import jax.numpy as jnp
from jax import lax
from jax.experimental import pallas as pl
from jax.experimental.pallas import tpu as pltpu
from jax.experimental.pallas import tpu_sc as plsc

jax.config.update("jax_enable_x64", True)

_L = 16

_MESH = plsc.VectorSubcoreMesh(core_axis_name="c", subcore_axis_name="s")


def _f32(word):
    return lax.bitcast_convert_type(word, jnp.float32)


def _srl(x, n):
    return lax.shift_right_logical(x, jnp.int32(n))


def _f32_to_f64_words(f):
    b = lax.bitcast_convert_type(f, jnp.int32)
    sign_top = _srl(b, 31) << 31
    e32 = _srl(b, 23) & 0xFF
    m = b & 0x7FFFFF
    hi = sign_top | ((e32 + 896) << 20) | _srl(m, 3)
    lo = m << 29
    hi = jnp.where(e32 > 0, hi, sign_top)
    lo = jnp.where(e32 > 0, lo, 0)
    return hi, lo


def _sc_body(src_hbm, ten_hbm, idx_hbm, out_hbm, src_v, ten_v, idx_v, out_v):
    c = lax.axis_index("c")
    s = lax.axis_index("s")

    @pl.when(jnp.logical_and(c == 0, s == 0))
    def _():
        pltpu.sync_copy(src_hbm, src_v.at[pl.ds(0, 2)])
        pltpu.sync_copy(ten_hbm, ten_v.at[pl.ds(0, 2)])
        pltpu.sync_copy(idx_hbm, idx_v.at[pl.ds(0, 2)])

        sv = src_v[...]
        tv = ten_v[...]
        iv = idx_v[...]
        src_f = _f32(sv[1]) + _f32(sv[0])
        ten_f = _f32(tv[1]) + _f32(tv[0])
        idx_is_zero = (iv[0] | iv[1]) == 0

        out_f = ten_f * jnp.float32(0.5) + jnp.where(
            idx_is_zero, src_f, jnp.float32(0.0))

        hi, lo = _f32_to_f64_words(out_f)
        lanes = lax.iota(jnp.int32, _L)
        out_v[...] = jnp.where(lanes == 1, hi, lo)
        pltpu.sync_copy(out_v.at[pl.ds(0, 2)], out_hbm)


def _scatter_add_halve(src_b, ten_b, idx_b):
    run = pl.kernel(
        _sc_body,
        out_type=jax.ShapeDtypeStruct((2,), jnp.int32),
        mesh=_MESH,
        scratch_types=[
            pltpu.VMEM((_L,), jnp.int32),
            pltpu.VMEM((_L,), jnp.int32),
            pltpu.VMEM((_L,), jnp.int32),
            pltpu.VMEM((_L,), jnp.int32),
        ],
    )
    return run(src_b, ten_b, idx_b)


def kernel(source, tensor, index):
    src_b = lax.bitcast_convert_type(source, jnp.int32).reshape(2)
    ten_b = lax.bitcast_convert_type(tensor, jnp.int32).reshape(2)
    idx_b = lax.bitcast_convert_type(index, jnp.int32).reshape(2)
    out_b = _scatter_add_halve(src_b, ten_b, idx_b)
    out = lax.bitcast_convert_type(out_b.reshape(1, 2), jnp.float64)
    return (source, out)


# speedup vs baseline: 1.5104x; 1.5104x over previous
import jax
import jax.numpy as jnp
from jax import lax
from jax.experimental import pallas as pl
from jax.experimental.pallas import tpu as pltpu
from jax.experimental.pallas import tpu_sc as plsc

jax.config.update("jax_enable_x64", True)

_L = 16

_MESH = plsc.VectorSubcoreMesh(core_axis_name="c", subcore_axis_name="s")


def _sc_body(src_hbm, ten_hbm, idx_hbm, out_hbm, src_v, ten_v, idx_v, out_v):
    c = lax.axis_index("c")
    s = lax.axis_index("s")

    @pl.when(jnp.logical_and(c == 0, s == 0))
    def _():
        pltpu.sync_copy(src_hbm, src_v.at[pl.ds(0, 1)])
        pltpu.sync_copy(ten_hbm, ten_v.at[pl.ds(0, 1)])
        pltpu.sync_copy(idx_hbm, idx_v.at[pl.ds(0, 1)])

        src_f = src_v[...][0]
        ten_f = ten_v[...][0]
        idx = idx_v[...][0]

        out_f = ten_f * jnp.float32(0.5) + jnp.where(
            idx == 0, src_f, jnp.float32(0.0))

        lanes = lax.iota(jnp.int32, _L)
        out_v[...] = jnp.where(lanes == 0, out_f, jnp.float32(0.0))
        pltpu.sync_copy(out_v.at[pl.ds(0, 1)], out_hbm)


def _scatter_add_halve(src32, ten32, idx32):
    run = pl.kernel(
        _sc_body,
        out_type=jax.ShapeDtypeStruct((1,), jnp.float32),
        mesh=_MESH,
        scratch_types=[
            pltpu.VMEM((_L,), jnp.float32),
            pltpu.VMEM((_L,), jnp.float32),
            pltpu.VMEM((_L,), jnp.int32),
            pltpu.VMEM((_L,), jnp.float32),
        ],
    )
    return run(src32, ten32, idx32)


def kernel(source, tensor, index):
    src32 = source.astype(jnp.float32)
    ten32 = tensor.astype(jnp.float32)
    idx32 = index.astype(jnp.int32)
    out = _scatter_add_halve(src32, ten32, idx32).astype(jnp.float64)
    return (source, out)


# speedup vs baseline: 1.6752x; 1.1091x over previous
import jax
import jax.numpy as jnp
from jax import lax
from jax.experimental import pallas as pl
from jax.experimental.pallas import tpu as pltpu
from jax.experimental.pallas import tpu_sc as plsc

jax.config.update("jax_enable_x64", True)

_L = 16

_MESH = plsc.VectorSubcoreMesh(core_axis_name="c", subcore_axis_name="s",
                               num_cores=1, num_subcores=1)


def _sc_body(src_hbm, ten_hbm, idx_hbm, out_hbm,
             src_v, ten_v, idx_v, out_v, sem0, sem1, sem2):
    c1 = pltpu.async_copy(src_hbm, src_v.at[pl.ds(0, 1)], sem0)
    c2 = pltpu.async_copy(ten_hbm, ten_v.at[pl.ds(0, 1)], sem1)
    c3 = pltpu.async_copy(idx_hbm, idx_v.at[pl.ds(0, 1)], sem2)
    c1.wait()
    c2.wait()
    c3.wait()

    src_f = src_v[...][0]
    ten_f = ten_v[...][0]
    idx = idx_v[...][0]

    out_f = ten_f * jnp.float32(0.5) + jnp.where(
        idx == 0, src_f, jnp.float32(0.0))

    lanes = lax.iota(jnp.int32, _L)
    out_v[...] = jnp.where(lanes == 0, out_f, jnp.float32(0.0))
    pltpu.sync_copy(out_v.at[pl.ds(0, 1)], out_hbm)


def _scatter_add_halve(src32, ten32, idx32):
    run = pl.kernel(
        _sc_body,
        out_type=jax.ShapeDtypeStruct((1,), jnp.float32),
        mesh=_MESH,
        scratch_types=[
            pltpu.VMEM((_L,), jnp.float32),
            pltpu.VMEM((_L,), jnp.float32),
            pltpu.VMEM((_L,), jnp.uint32),
            pltpu.VMEM((_L,), jnp.float32),
            pltpu.SemaphoreType.DMA,
            pltpu.SemaphoreType.DMA,
            pltpu.SemaphoreType.DMA,
        ],
    )
    return run(src32, ten32, idx32)


def kernel(source, tensor, index):
    src32 = source.astype(jnp.float32)
    ten32 = tensor.astype(jnp.float32)
    idx32 = index.astype(jnp.uint32)
    out = _scatter_add_halve(src32, ten32, idx32).astype(jnp.float64)
    return (source, out)
